# baseline (speedup 1.0000x reference)
import functools

import jax
import jax.numpy as jnp
from jax import lax
from jax.experimental import pallas as pl
from jax.experimental.pallas import tpu as pltpu
from jax.experimental.pallas import tpu_sc as plsc

B = 16
T = 50
P = 8
H = 64
W = 64
N = B * T
NC = 2
NS = 16
NWORK = NC * NS
NSC = 120
WRK = 30
RPW = NSC // WRK
NTC = N - NSC
G = 40


def _sc_body(xd_h, yd_h, dx_h, dy_h, ox_h, oy_h, z_h, out_h,
             xv, yv, dxv, dyv, oxv, oyv, buf0, buf1, sem0, sem1):
    w = lax.axis_index("s") * NC + lax.axis_index("c")

    @pl.when(w < WRK)
    def _():
        _worker(w, xd_h, yd_h, dx_h, dy_h, ox_h, oy_h, z_h, out_h,
                xv, yv, dxv, dyv, oxv, oyv, buf0, buf1, sem0, sem1)


def _worker(w, xd_h, yd_h, dx_h, dy_h, ox_h, oy_h, z_h, out_h,
            xv, yv, dxv, dyv, oxv, oyv, buf0, buf1, sem0, sem1):
    base = w * RPW

    pltpu.sync_copy(xd_h.at[pl.ds(base * 16, RPW * 16)], xv)
    pltpu.sync_copy(yd_h.at[pl.ds(base * 16, RPW * 16)], yv)
    pltpu.sync_copy(dx_h.at[pl.ds(base * 16, RPW * 16)], dxv)
    pltpu.sync_copy(dy_h.at[pl.ds(base * 16, RPW * 16)], dyv)
    pltpu.sync_copy(ox_h.at[pl.ds(base * 16, RPW * 16)], oxv)
    pltpu.sync_copy(oy_h.at[pl.ds(base * 16, RPW * 16)], oyv)
    pltpu.sync_copy(z_h, buf0)
    pltpu.sync_copy(z_h, buf1)

    lane = lax.iota(jnp.int32, 16)
    lane_p = lane & 7
    mask_lo = lane < 8
    ones = jnp.full((16,), 1.0, jnp.float32)
    zeros_v = jnp.zeros((16,), jnp.float32)

    bufs = (buf0, buf1)
    sems = (sem0, sem1)
    prev = [None, None]
    handles = [None] * (2 * RPW)
    ok = riq = ciq = None
    for hs in range(2 * RPW):
        s, q = hs >> 1, hs & 1
        b = hs & 1
        buf = bufs[b]
        if hs >= 2:
            handles[hs - 2].wait()
            idx_old, msk_old = prev[b]
            plsc.store_scatter(buf, idx_old, zeros_v, mask=msk_old)
        if q == 0:
            sl = pl.ds(s * 16, 16)
            cf = xv[sl] / dxv[sl] + oxv[sl]
            rf = yv[sl] / dyv[sl] + oyv[sl]
            ci = cf.astype(jnp.int32)
            ri = rf.astype(jnp.int32)
            ok = mask_lo & (ci >= 0) & (ci < W) & (ri >= 0) & (ri < H)
            ciq = jnp.clip(ci, 0, W - 1)
            riq = jnp.clip(ri, 0, H - 1)
        okq = ok & ((riq >= q * (H // 2)) & (riq < (q + 1) * (H // 2)))
        rloc = jnp.clip(riq - q * (H // 2), 0, H // 2 - 1)
        idx = [rloc, lane_p, ciq]
        plsc.store_scatter(buf, idx, ones, mask=okq)
        handles[hs] = pltpu.async_copy(
            buf, out_h.at[base + s, pl.ds(q * (H // 2), H // 2)], sems[b])
        prev[b] = (idx, okq)
    handles[2 * RPW - 2].wait()
    handles[2 * RPW - 1].wait()


_sc_fn = functools.partial(
    pl.kernel,
    out_type=jax.ShapeDtypeStruct((N, H, P, W), jnp.float32),
    mesh=plsc.VectorSubcoreMesh(core_axis_name="c", subcore_axis_name="s"),
    compiler_params=pltpu.CompilerParams(needs_layout_passes=False),
    scratch_types=[
        pltpu.VMEM((RPW * 16,), jnp.float32),
        pltpu.VMEM((RPW * 16,), jnp.float32),
        pltpu.VMEM((RPW * 16,), jnp.float32),
        pltpu.VMEM((RPW * 16,), jnp.float32),
        pltpu.VMEM((RPW * 16,), jnp.float32),
        pltpu.VMEM((RPW * 16,), jnp.float32),
        pltpu.VMEM((H // 2, P, W), jnp.float32),
        pltpu.VMEM((H // 2, P, W), jnp.float32),
        pltpu.SemaphoreType.DMA,
        pltpu.SemaphoreType.DMA,
    ],
)(_sc_body)


def _tc_body(acc_ref, xr, yr, resr, orgr, out_ref):
    del acc_ref
    res = resr[...]
    org = orgr[...]
    coli = (xr[...] / res[:, 0:1] + org[:, 1:2]).astype(jnp.int32)
    rowi = (yr[...] / res[:, 1:2] + org[:, 0:1]).astype(jnp.int32)
    inb = (coli >= 0) & (coli < W) & (rowi >= 0) & (rowi < H)
    tgt_r = jnp.where(inb, rowi, -1)
    hio = jax.lax.broadcasted_iota(jnp.int32, (G, H, P, W), 1)
    wio = jax.lax.broadcasted_iota(jnp.int32, (G, H, P, W), 3)
    hit = (hio == tgt_r[:, None, :, None]) & (wio == coli[:, None, :, None])
    out_ref[...] = hit.astype(jnp.float32)


def kernel(x, resolution, origin):
    pts = x.reshape(N, P, 2)
    xc = pts[:, :, 0]
    yc = pts[:, :, 1]
    res = resolution.reshape(N, 2)
    org = origin.reshape(N, 2)

    xd = jnp.tile(xc[:NSC], (1, 2)).reshape(-1)
    yd = jnp.tile(yc[:NSC], (1, 2)).reshape(-1)
    dx = jnp.tile(res[:NSC, 0:1], (1, 16)).reshape(-1)
    dy = jnp.tile(res[:NSC, 1:2], (1, 16)).reshape(-1)
    ox = jnp.tile(org[:NSC, 1:2], (1, 16)).reshape(-1)
    oy = jnp.tile(org[:NSC, 0:1], (1, 16)).reshape(-1)
    z = jnp.zeros((H // 2, P, W), jnp.float32)
    grid_sc = _sc_fn(xd, yd, dx, dy, ox, oy, z)

    off = NSC // G
    out = pl.pallas_call(
        _tc_body,
        grid=(NTC // G,),
        in_specs=[
            pl.BlockSpec(memory_space=pl.ANY),
            pl.BlockSpec((G, P), lambda i: (i + off, 0)),
            pl.BlockSpec((G, P), lambda i: (i + off, 0)),
            pl.BlockSpec((G, 2), lambda i: (i + off, 0)),
            pl.BlockSpec((G, 2), lambda i: (i + off, 0)),
        ],
        out_specs=pl.BlockSpec((G, H, P, W), lambda i: (i + off, 0, 0, 0)),
        out_shape=jax.ShapeDtypeStruct((N, H, P, W), jnp.float32),
        input_output_aliases={0: 0},
    )(grid_sc, xc, yc, res, org)

    out5 = out.reshape(B, T, H, P, W)
    return jnp.transpose(out5, (0, 1, 2, 4, 3))


# speedup vs baseline: 1.0509x; 1.0509x over previous
import functools

import jax
import jax.numpy as jnp
from jax import lax
from jax.experimental import pallas as pl
from jax.experimental.pallas import tpu as pltpu
from jax.experimental.pallas import tpu_sc as plsc

B = 16
T = 50
P = 8
H = 64
W = 64
N = B * T
NC = 2
NS = 16
NWORK = NC * NS
NSC = 40
WRK = 20
RPW = NSC // WRK
NTC = N - NSC
G = 40


def _sc_body(xd_h, yd_h, dx_h, dy_h, ox_h, oy_h, z_h, out_h,
             xv, yv, dxv, dyv, oxv, oyv, buf0, buf1, sem0, sem1):
    w = lax.axis_index("s") * NC + lax.axis_index("c")

    @pl.when(w < WRK)
    def _():
        _worker(w, xd_h, yd_h, dx_h, dy_h, ox_h, oy_h, z_h, out_h,
                xv, yv, dxv, dyv, oxv, oyv, buf0, buf1, sem0, sem1)


def _worker(w, xd_h, yd_h, dx_h, dy_h, ox_h, oy_h, z_h, out_h,
            xv, yv, dxv, dyv, oxv, oyv, buf0, buf1, sem0, sem1):
    base = w * RPW

    pltpu.sync_copy(xd_h.at[pl.ds(base * 16, RPW * 16)], xv)
    pltpu.sync_copy(yd_h.at[pl.ds(base * 16, RPW * 16)], yv)
    pltpu.sync_copy(dx_h.at[pl.ds(base * 16, RPW * 16)], dxv)
    pltpu.sync_copy(dy_h.at[pl.ds(base * 16, RPW * 16)], dyv)
    pltpu.sync_copy(ox_h.at[pl.ds(base * 16, RPW * 16)], oxv)
    pltpu.sync_copy(oy_h.at[pl.ds(base * 16, RPW * 16)], oyv)
    pltpu.sync_copy(z_h, buf0)
    pltpu.sync_copy(z_h, buf1)

    lane = lax.iota(jnp.int32, 16)
    lane_p = lane & 7
    mask_lo = lane < 8
    ones = jnp.full((16,), 1.0, jnp.float32)
    zeros_v = jnp.zeros((16,), jnp.float32)

    bufs = (buf0, buf1)
    sems = (sem0, sem1)
    prev = [None, None]
    handles = [None] * (2 * RPW)
    ok = riq = ciq = None
    for hs in range(2 * RPW):
        s, q = hs >> 1, hs & 1
        b = hs & 1
        buf = bufs[b]
        if hs >= 2:
            handles[hs - 2].wait()
            idx_old, msk_old = prev[b]
            plsc.store_scatter(buf, idx_old, zeros_v, mask=msk_old)
        if q == 0:
            sl = pl.ds(s * 16, 16)
            cf = xv[sl] / dxv[sl] + oxv[sl]
            rf = yv[sl] / dyv[sl] + oyv[sl]
            ci = cf.astype(jnp.int32)
            ri = rf.astype(jnp.int32)
            ok = mask_lo & (ci >= 0) & (ci < W) & (ri >= 0) & (ri < H)
            ciq = jnp.clip(ci, 0, W - 1)
            riq = jnp.clip(ri, 0, H - 1)
        okq = ok & ((riq >= q * (H // 2)) & (riq < (q + 1) * (H // 2)))
        rloc = jnp.clip(riq - q * (H // 2), 0, H // 2 - 1)
        idx = [rloc, lane_p, ciq]
        plsc.store_scatter(buf, idx, ones, mask=okq)
        handles[hs] = pltpu.async_copy(
            buf, out_h.at[base + s, pl.ds(q * (H // 2), H // 2)], sems[b])
        prev[b] = (idx, okq)
    handles[2 * RPW - 2].wait()
    handles[2 * RPW - 1].wait()


_sc_fn = functools.partial(
    pl.kernel,
    out_type=jax.ShapeDtypeStruct((N, H, P, W), jnp.float32),
    mesh=plsc.VectorSubcoreMesh(core_axis_name="c", subcore_axis_name="s"),
    compiler_params=pltpu.CompilerParams(needs_layout_passes=False),
    scratch_types=[
        pltpu.VMEM((RPW * 16,), jnp.float32),
        pltpu.VMEM((RPW * 16,), jnp.float32),
        pltpu.VMEM((RPW * 16,), jnp.float32),
        pltpu.VMEM((RPW * 16,), jnp.float32),
        pltpu.VMEM((RPW * 16,), jnp.float32),
        pltpu.VMEM((RPW * 16,), jnp.float32),
        pltpu.VMEM((H // 2, P, W), jnp.float32),
        pltpu.VMEM((H // 2, P, W), jnp.float32),
        pltpu.SemaphoreType.DMA,
        pltpu.SemaphoreType.DMA,
    ],
)(_sc_body)


def _tc_body(acc_ref, xr, yr, resr, orgr, out_ref):
    del acc_ref
    res = resr[...]
    org = orgr[...]
    coli = (xr[...] / res[:, 0:1] + org[:, 1:2]).astype(jnp.int32)
    rowi = (yr[...] / res[:, 1:2] + org[:, 0:1]).astype(jnp.int32)
    inb = (coli >= 0) & (coli < W) & (rowi >= 0) & (rowi < H)
    tgt_r = jnp.where(inb, rowi, -1)
    hio = jax.lax.broadcasted_iota(jnp.int32, (G, H, P, W), 1)
    wio = jax.lax.broadcasted_iota(jnp.int32, (G, H, P, W), 3)
    hit = (hio == tgt_r[:, None, :, None]) & (wio == coli[:, None, :, None])
    out_ref[...] = hit.astype(jnp.float32)


def kernel(x, resolution, origin):
    pts = x.reshape(N, P, 2)
    xc = pts[:, :, 0]
    yc = pts[:, :, 1]
    res = resolution.reshape(N, 2)
    org = origin.reshape(N, 2)

    xd = jnp.tile(xc[:NSC], (1, 2)).reshape(-1)
    yd = jnp.tile(yc[:NSC], (1, 2)).reshape(-1)
    dx = jnp.tile(res[:NSC, 0:1], (1, 16)).reshape(-1)
    dy = jnp.tile(res[:NSC, 1:2], (1, 16)).reshape(-1)
    ox = jnp.tile(org[:NSC, 1:2], (1, 16)).reshape(-1)
    oy = jnp.tile(org[:NSC, 0:1], (1, 16)).reshape(-1)
    z = jnp.zeros((H // 2, P, W), jnp.float32)
    grid_sc = _sc_fn(xd, yd, dx, dy, ox, oy, z)

    off = 1
    out = pl.pallas_call(
        _tc_body,
        grid=(NTC // G,),
        in_specs=[
            pl.BlockSpec(memory_space=pl.ANY),
            pl.BlockSpec((G, P), lambda i: (i + off, 0)),
            pl.BlockSpec((G, P), lambda i: (i + off, 0)),
            pl.BlockSpec((G, 2), lambda i: (i + off, 0)),
            pl.BlockSpec((G, 2), lambda i: (i + off, 0)),
        ],
        out_specs=pl.BlockSpec((G, H, P, W), lambda i: (i + off, 0, 0, 0)),
        out_shape=jax.ShapeDtypeStruct((N, H, P, W), jnp.float32),
        input_output_aliases={0: 0},
    )(grid_sc, xc, yc, res, org)

    out5 = out.reshape(B, T, H, P, W)
    return jnp.transpose(out5, (0, 1, 2, 4, 3))
